# baseline (speedup 1.0000x reference)
import functools

import jax
import jax.numpy as jnp
from jax import lax
from jax.experimental import pallas as pl
from jax.experimental.pallas import tpu as pltpu
from jax.experimental.pallas import tpu_sc as plsc

N_NODES = 10000
N_EDGES = 160000
C_IN = 128
C_OUT = 16
EDGE_DIM = 16

NC = 2
NS = 16
NW = NC * NS
EDGES_PER_WORKER = N_EDGES // NW
CHUNK = 125
CP = 128
NCHUNKS = EDGES_PER_WORKER // CHUNK
ROWS_PER_TILE = 640
N_PAD = NS * ROWS_PER_TILE


def _pre_body(nf_ref, w2a_ref, w2b_ref, b2_ref, ea_ref, w1_ref, b1_ref,
              rw_ref, b_ref, gba_ref, gbb_ref, bv_ref, h_ref, root_ref):
    nf = nf_ref[...]
    gba_ref[...] = jnp.dot(nf, w2a_ref[...],
                           preferred_element_type=jnp.float32
                           ).astype(jnp.bfloat16)
    gbb_ref[...] = jnp.dot(nf, w2b_ref[...],
                           preferred_element_type=jnp.float32
                           ).astype(jnp.bfloat16)
    bv_ref[...] = jnp.dot(nf, b2_ref[...], preferred_element_type=jnp.float32)
    h_ref[...] = jnp.maximum(
        jnp.dot(ea_ref[...], w1_ref[...],
                preferred_element_type=jnp.float32) + b1_ref[...], 0.0)
    root_ref[...] = jnp.dot(nf, rw_ref[...],
                            preferred_element_type=jnp.float32) + b_ref[...]


def _combine_body(root_ref, p0_ref, p1_ref, o_ref):
    o_ref[...] = root_ref[...] + p0_ref[...] + p1_ref[...]


_mesh = plsc.VectorSubcoreMesh(core_axis_name="c", subcore_axis_name="s")


@functools.partial(
    pl.kernel,
    mesh=_mesh,
    compiler_params=pltpu.CompilerParams(use_tc_tiling_on_sc=False,
                                         needs_layout_passes=False),
    out_type=(jax.ShapeDtypeStruct((N_PAD, C_OUT), jnp.float32),
              jax.ShapeDtypeStruct((N_PAD, C_OUT), jnp.float32)),
    scratch_types=[
        pltpu.VMEM((NCHUNKS, CP), jnp.int32),
        pltpu.VMEM((NCHUNKS, CP), jnp.int32),
        pltpu.VMEM((CP, 128), jnp.bfloat16),
        pltpu.VMEM((CP, 128), jnp.bfloat16),
        pltpu.VMEM((CP, 128), jnp.bfloat16),
        pltpu.VMEM((CP, 128), jnp.bfloat16),
        pltpu.VMEM((CP, C_OUT), jnp.float32),
        pltpu.VMEM((CP, C_OUT), jnp.float32),
        pltpu.VMEM((CHUNK * C_OUT,), jnp.float32),
        pltpu.VMEM((CHUNK * C_OUT,), jnp.float32),
        pltpu.VMEM((CP, C_OUT), jnp.float32),
        pltpu.VMEM((CP, C_OUT), jnp.float32),
        pltpu.VMEM((ROWS_PER_TILE, C_OUT), jnp.float32),
        pltpu.VMEM_SHARED((N_PAD, C_OUT), jnp.float32),
        pltpu.SemaphoreType.DMA,
        pltpu.SemaphoreType.DMA,
        pltpu.SemaphoreType.DMA,
        pltpu.SemaphoreType.DMA,
    ],
)
def _sc_edge_kernel(gba_hbm, gbb_hbm, bv_hbm, h_hbm, src_hbm, dst_hbm,
                    out0_hbm, out1_hbm,
                    src_v, dst_v, rowsa0, rowsa1, rowsb0, rowsb1, bv0, bv1,
                    h0, h1, msg0, msg1, z_v, acc_sh,
                    semg0, semg1, sems0, sems1):
    c = lax.axis_index("c")
    s = lax.axis_index("s")
    wid = s * NC + c
    rowsa = (rowsa0, rowsa1)
    rowsb = (rowsb0, rowsb1)
    bv = (bv0, bv1)
    hb = (h0, h1)
    msg = (msg0, msg1)
    semg = (semg0, semg1)
    sems = (sems0, sems1)

    def _zero_row(i, carry):
        z_v[i, :] = jnp.zeros((C_OUT,), jnp.float32)
        return carry

    lax.fori_loop(0, ROWS_PER_TILE, _zero_row, 0)
    pltpu.sync_copy(z_v, acc_sh.at[pl.ds(s * ROWS_PER_TILE, ROWS_PER_TILE)])
    plsc.subcore_barrier()

    for b in (0, 1):
        for i in range(CHUNK, CP):
            msg[b][i, :] = jnp.zeros((C_OUT,), jnp.float32)

    pltpu.sync_copy(src_hbm.at[wid], src_v)
    pltpu.sync_copy(dst_hbm.at[wid], dst_v)

    hbase = wid * EDGES_PER_WORKER * C_OUT

    def _issue(j, b):
        idx = src_v.at[j]
        pltpu.async_copy(gba_hbm.at[idx], rowsa[b], semg[b])
        pltpu.async_copy(gbb_hbm.at[idx], rowsb[b], semg[b])
        pltpu.async_copy(bv_hbm.at[idx], bv[b], semg[b])
        pltpu.async_copy(h_hbm.at[pl.ds(hbase + j * (CHUNK * C_OUT),
                                        CHUNK * C_OUT)], hb[b], semg[b])

    def _drain(j, b):
        idx = src_v.at[j]
        pltpu.make_async_copy(gba_hbm.at[idx], rowsa[b], semg[b]).wait()
        pltpu.make_async_copy(gbb_hbm.at[idx], rowsb[b], semg[b]).wait()
        pltpu.make_async_copy(bv_hbm.at[idx], bv[b], semg[b]).wait()
        pltpu.make_async_copy(h_hbm.at[pl.ds(hbase + j * (CHUNK * C_OUT),
                                             CHUNK * C_OUT)],
                              hb[b], semg[b]).wait()

    def _process(t, j, b):
        _drain(j, b)

        @pl.when(t > 0)
        def _():
            pltpu.make_async_copy(msg[b], acc_sh.at[dst_v.at[j]],
                                  sems[b]).wait()

        def _edge(e, ecarry):
            hv = hb[b][pl.ds(e * C_OUT, C_OUT)]
            acc = bv[b][e, :]
            for g in range(4):
                va = rowsa[b][e, pl.ds(g * 32, 32)]
                a0, a1 = plsc.unpack(va, format=plsc.PackFormat.INTERLEAVED)
                acc = acc + hv[2 * g] * a0 + hv[2 * g + 1] * a1
            for g in range(4):
                vb2 = rowsb[b][e, pl.ds(g * 32, 32)]
                b0, b1v = plsc.unpack(vb2, format=plsc.PackFormat.INTERLEAVED)
                acc = acc + hv[8 + 2 * g] * b0 + hv[8 + 2 * g + 1] * b1v
            msg[b][e, :] = acc
            return ecarry

        lax.fori_loop(0, CHUNK, _edge, 0, unroll=5)
        pltpu.async_copy(msg[b], acc_sh.at[dst_v.at[j]], sems[b], add=True)

    _issue(0, 0)

    def _super(t, carry):
        _issue(2 * t + 1, 1)
        _process(t, 2 * t, 0)

        @pl.when(t < NCHUNKS // 2 - 1)
        def _():
            _issue(2 * t + 2, 0)

        _process(t, 2 * t + 1, 1)
        return carry

    lax.fori_loop(0, NCHUNKS // 2, _super, 0)

    pltpu.make_async_copy(msg[0], acc_sh.at[dst_v.at[0]], sems[0]).wait()
    pltpu.make_async_copy(msg[1], acc_sh.at[dst_v.at[0]], sems[1]).wait()
    plsc.subcore_barrier()

    row0 = s * ROWS_PER_TILE

    @pl.when(c == 0)
    def _():
        pltpu.sync_copy(acc_sh.at[pl.ds(row0, ROWS_PER_TILE)],
                        out0_hbm.at[pl.ds(row0, ROWS_PER_TILE)])

    @pl.when(c == 1)
    def _():
        pltpu.sync_copy(acc_sh.at[pl.ds(row0, ROWS_PER_TILE)],
                        out1_hbm.at[pl.ds(row0, ROWS_PER_TILE)])


def kernel(node_feats, edge_attr, W1, b1, W2, b2, root_W, bias, edge_index):
    w2t = W2.reshape(EDGE_DIM, C_IN, C_OUT).transpose(1, 0, 2)

    def _interleave(half):
        return half.reshape(C_IN, 4, 2, C_OUT).transpose(0, 1, 3, 2).reshape(
            C_IN, 128)

    W2Pa = _interleave(w2t[:, :8, :])
    W2Pb = _interleave(w2t[:, 8:, :])
    b2r = b2.reshape(C_IN, C_OUT)

    src = jnp.pad(edge_index[0].reshape(NW, NCHUNKS, CHUNK),
                  ((0, 0), (0, 0), (0, CP - CHUNK)))
    dst = jnp.pad(edge_index[1].reshape(NW, NCHUNKS, CHUNK),
                  ((0, 0), (0, 0), (0, CP - CHUNK)))

    ea_wide = edge_attr.reshape(N_EDGES // 8, 8 * EDGE_DIM)
    W1blk = jnp.kron(jnp.eye(8, dtype=W1.dtype), W1)
    b1w = jnp.tile(b1, 8).reshape(1, 8 * C_OUT)

    ng = 5
    bn = N_NODES // ng
    bew = N_EDGES // 8 // ng
    gba, gbb, bvals, hw, root = pl.pallas_call(
        _pre_body,
        grid=(ng,),
        in_specs=[pl.BlockSpec((bn, C_IN), lambda i: (i, 0)),
                  pl.BlockSpec((C_IN, 128), lambda i: (0, 0)),
                  pl.BlockSpec((C_IN, 128), lambda i: (0, 0)),
                  pl.BlockSpec((C_IN, C_OUT), lambda i: (0, 0)),
                  pl.BlockSpec((bew, 128), lambda i: (i, 0)),
                  pl.BlockSpec((128, 128), lambda i: (0, 0)),
                  pl.BlockSpec((1, 128), lambda i: (0, 0)),
                  pl.BlockSpec((C_IN, C_OUT), lambda i: (0, 0)),
                  pl.BlockSpec((1, C_OUT), lambda i: (0, 0))],
        out_specs=[pl.BlockSpec((bn, 128), lambda i: (i, 0)),
                   pl.BlockSpec((bn, 128), lambda i: (i, 0)),
                   pl.BlockSpec((bn, C_OUT), lambda i: (i, 0)),
                   pl.BlockSpec((bew, 128), lambda i: (i, 0)),
                   pl.BlockSpec((bn, C_OUT), lambda i: (i, 0))],
        out_shape=[jax.ShapeDtypeStruct((N_NODES, 128), jnp.bfloat16),
                   jax.ShapeDtypeStruct((N_NODES, 128), jnp.bfloat16),
                   jax.ShapeDtypeStruct((N_NODES, C_OUT), jnp.float32),
                   jax.ShapeDtypeStruct((N_EDGES // 8, 128), jnp.float32),
                   jax.ShapeDtypeStruct((N_NODES, C_OUT), jnp.float32)],
    )(node_feats, W2Pa, W2Pb, b2r, ea_wide, W1blk, b1w, root_W,
      bias.reshape(1, C_OUT))

    h_flat = hw.reshape(N_EDGES * C_OUT)
    p0, p1 = _sc_edge_kernel(gba, gbb, bvals, h_flat, src, dst)

    bc = 2000
    out = pl.pallas_call(
        _combine_body,
        grid=(N_NODES // bc,),
        in_specs=[pl.BlockSpec((bc, C_OUT), lambda i: (i, 0)),
                  pl.BlockSpec((bc, C_OUT), lambda i: (i, 0)),
                  pl.BlockSpec((bc, C_OUT), lambda i: (i, 0))],
        out_specs=pl.BlockSpec((bc, C_OUT), lambda i: (i, 0)),
        out_shape=jax.ShapeDtypeStruct((N_NODES, C_OUT), jnp.float32),
    )(root, p0, p1)

    return out


# speedup vs baseline: 1.0358x; 1.0358x over previous
import functools

import jax
import jax.numpy as jnp
import numpy as np
from jax import lax
from jax.experimental import pallas as pl
from jax.experimental.pallas import tpu as pltpu
from jax.experimental.pallas import tpu_sc as plsc

N_NODES = 10000
N_EDGES = 160000
C_IN = 128
C_OUT = 16
EDGE_DIM = 16

NC = 2
NS = 16
NW = NC * NS
EDGES_PER_WORKER = N_EDGES // NW
CHUNK = 125
CP = 128
NCHUNKS = EDGES_PER_WORKER // CHUNK
ROWS_PER_TILE = 640
N_PAD = NS * ROWS_PER_TILE

_HI_MASK = np.int32(-65536)
_LO_MASK = np.int32(0xFFFF)


def _round_bf16_bits(x):
    b = lax.bitcast_convert_type(x, jnp.int32)
    return b + np.int32(0x7FFF) + ((b >> 16) & np.int32(1))


def _pre_body(nf_ref, w2l_ref, w2h_ref, b2_ref, ea_ref, w1_ref, b1_ref,
              rw_ref, b_ref, gbp_ref, bv_ref, h_ref, root_ref):
    nf = nf_ref[...]
    lo = jnp.dot(nf, w2l_ref[...], preferred_element_type=jnp.float32)
    hi = jnp.dot(nf, w2h_ref[...], preferred_element_type=jnp.float32)
    rl = _round_bf16_bits(lo)
    rh = _round_bf16_bits(hi)
    gbp_ref[...] = (rh & _HI_MASK) | ((rl >> 16) & _LO_MASK)
    bv_ref[...] = jnp.dot(nf, b2_ref[...], preferred_element_type=jnp.float32)
    h_ref[...] = jnp.maximum(
        jnp.dot(ea_ref[...], w1_ref[...],
                preferred_element_type=jnp.float32) + b1_ref[...], 0.0)
    root_ref[...] = jnp.dot(nf, rw_ref[...],
                            preferred_element_type=jnp.float32) + b_ref[...]


def _combine_body(root_ref, p0_ref, p1_ref, o_ref):
    o_ref[...] = root_ref[...] + p0_ref[...] + p1_ref[...]


_mesh = plsc.VectorSubcoreMesh(core_axis_name="c", subcore_axis_name="s")


@functools.partial(
    pl.kernel,
    mesh=_mesh,
    compiler_params=pltpu.CompilerParams(use_tc_tiling_on_sc=False,
                                         needs_layout_passes=False),
    out_type=(jax.ShapeDtypeStruct((N_PAD, C_OUT), jnp.float32),
              jax.ShapeDtypeStruct((N_PAD, C_OUT), jnp.float32)),
    scratch_types=[
        pltpu.VMEM((NCHUNKS, CP), jnp.int32),
        pltpu.VMEM((NCHUNKS, CP), jnp.int32),
        pltpu.VMEM((CP, 128), jnp.int32),
        pltpu.VMEM((CP, 128), jnp.int32),
        pltpu.VMEM((CP, C_OUT), jnp.float32),
        pltpu.VMEM((CP, C_OUT), jnp.float32),
        pltpu.VMEM((CHUNK * C_OUT,), jnp.float32),
        pltpu.VMEM((CHUNK * C_OUT,), jnp.float32),
        pltpu.VMEM((CP, C_OUT), jnp.float32),
        pltpu.VMEM((CP, C_OUT), jnp.float32),
        pltpu.VMEM((ROWS_PER_TILE, C_OUT), jnp.float32),
        pltpu.VMEM_SHARED((N_PAD, C_OUT), jnp.float32),
        pltpu.SemaphoreType.DMA,
        pltpu.SemaphoreType.DMA,
        pltpu.SemaphoreType.DMA,
        pltpu.SemaphoreType.DMA,
    ],
)
def _sc_edge_kernel(gbp_hbm, bv_hbm, h_hbm, src_hbm, dst_hbm,
                    out0_hbm, out1_hbm,
                    src_v, dst_v, rows0, rows1, bv0, bv1,
                    h0, h1, msg0, msg1, z_v, acc_sh,
                    semg0, semg1, sems0, sems1):
    c = lax.axis_index("c")
    s = lax.axis_index("s")
    wid = s * NC + c
    rows = (rows0, rows1)
    bv = (bv0, bv1)
    hb = (h0, h1)
    msg = (msg0, msg1)
    semg = (semg0, semg1)
    sems = (sems0, sems1)

    def _zero_row(i, carry):
        z_v[i, :] = jnp.zeros((C_OUT,), jnp.float32)
        return carry

    lax.fori_loop(0, ROWS_PER_TILE, _zero_row, 0)
    pltpu.sync_copy(z_v, acc_sh.at[pl.ds(s * ROWS_PER_TILE, ROWS_PER_TILE)])
    plsc.subcore_barrier()

    for b in (0, 1):
        for i in range(CHUNK, CP):
            msg[b][i, :] = jnp.zeros((C_OUT,), jnp.float32)

    pltpu.sync_copy(src_hbm.at[wid], src_v)
    pltpu.sync_copy(dst_hbm.at[wid], dst_v)

    hbase = wid * EDGES_PER_WORKER * C_OUT

    def _issue(j, b):
        idx = src_v.at[j]
        pltpu.async_copy(gbp_hbm.at[idx], rows[b], semg[b])
        pltpu.async_copy(bv_hbm.at[idx], bv[b], semg[b])
        pltpu.async_copy(h_hbm.at[pl.ds(hbase + j * (CHUNK * C_OUT),
                                        CHUNK * C_OUT)], hb[b], semg[b])

    def _drain(j, b):
        idx = src_v.at[j]
        pltpu.make_async_copy(gbp_hbm.at[idx], rows[b], semg[b]).wait()
        pltpu.make_async_copy(bv_hbm.at[idx], bv[b], semg[b]).wait()
        pltpu.make_async_copy(h_hbm.at[pl.ds(hbase + j * (CHUNK * C_OUT),
                                             CHUNK * C_OUT)],
                              hb[b], semg[b]).wait()

    def _process(t, j, b):
        _drain(j, b)

        @pl.when(t > 0)
        def _():
            pltpu.make_async_copy(msg[b], acc_sh.at[dst_v.at[j]],
                                  sems[b]).wait()

        def _edge(e, ecarry):
            hv = hb[b][pl.ds(e * C_OUT, C_OUT)]
            acc = bv[b][e, :]
            for q in range(8):
                v = rows[b][e, pl.ds(q * C_OUT, C_OUT)]
                lo = plsc.bitcast(v << 16, jnp.float32)
                hi = plsc.bitcast(v & _HI_MASK, jnp.float32)
                acc = acc + hv[2 * q] * lo + hv[2 * q + 1] * hi
            msg[b][e, :] = acc
            return ecarry

        lax.fori_loop(0, CHUNK, _edge, 0, unroll=5)
        pltpu.async_copy(msg[b], acc_sh.at[dst_v.at[j]], sems[b], add=True)

    _issue(0, 0)

    def _super(t, carry):
        _issue(2 * t + 1, 1)
        _process(t, 2 * t, 0)

        @pl.when(t < NCHUNKS // 2 - 1)
        def _():
            _issue(2 * t + 2, 0)

        _process(t, 2 * t + 1, 1)
        return carry

    lax.fori_loop(0, NCHUNKS // 2, _super, 0)

    pltpu.make_async_copy(msg[0], acc_sh.at[dst_v.at[0]], sems[0]).wait()
    pltpu.make_async_copy(msg[1], acc_sh.at[dst_v.at[0]], sems[1]).wait()
    plsc.subcore_barrier()

    row0 = s * ROWS_PER_TILE

    @pl.when(c == 0)
    def _():
        pltpu.sync_copy(acc_sh.at[pl.ds(row0, ROWS_PER_TILE)],
                        out0_hbm.at[pl.ds(row0, ROWS_PER_TILE)])

    @pl.when(c == 1)
    def _():
        pltpu.sync_copy(acc_sh.at[pl.ds(row0, ROWS_PER_TILE)],
                        out1_hbm.at[pl.ds(row0, ROWS_PER_TILE)])


def kernel(node_feats, edge_attr, W1, b1, W2, b2, root_W, bias, edge_index):
    w2t = W2.reshape(EDGE_DIM, C_IN, C_OUT).transpose(1, 0, 2)
    W2L = w2t[:, 0::2, :].reshape(C_IN, 128)
    W2H = w2t[:, 1::2, :].reshape(C_IN, 128)
    b2r = b2.reshape(C_IN, C_OUT)

    src = jnp.pad(edge_index[0].reshape(NW, NCHUNKS, CHUNK),
                  ((0, 0), (0, 0), (0, CP - CHUNK)))
    dst = jnp.pad(edge_index[1].reshape(NW, NCHUNKS, CHUNK),
                  ((0, 0), (0, 0), (0, CP - CHUNK)))

    ea_wide = edge_attr.reshape(N_EDGES // 8, 8 * EDGE_DIM)
    W1blk = jnp.kron(jnp.eye(8, dtype=W1.dtype), W1)
    b1w = jnp.tile(b1, 8).reshape(1, 8 * C_OUT)

    ng = 5
    bn = N_NODES // ng
    bew = N_EDGES // 8 // ng
    gbp, bvals, hw, root = pl.pallas_call(
        _pre_body,
        grid=(ng,),
        in_specs=[pl.BlockSpec((bn, C_IN), lambda i: (i, 0)),
                  pl.BlockSpec((C_IN, 128), lambda i: (0, 0)),
                  pl.BlockSpec((C_IN, 128), lambda i: (0, 0)),
                  pl.BlockSpec((C_IN, C_OUT), lambda i: (0, 0)),
                  pl.BlockSpec((bew, 128), lambda i: (i, 0)),
                  pl.BlockSpec((128, 128), lambda i: (0, 0)),
                  pl.BlockSpec((1, 128), lambda i: (0, 0)),
                  pl.BlockSpec((C_IN, C_OUT), lambda i: (0, 0)),
                  pl.BlockSpec((1, C_OUT), lambda i: (0, 0))],
        out_specs=[pl.BlockSpec((bn, 128), lambda i: (i, 0)),
                   pl.BlockSpec((bn, C_OUT), lambda i: (i, 0)),
                   pl.BlockSpec((bew, 128), lambda i: (i, 0)),
                   pl.BlockSpec((bn, C_OUT), lambda i: (i, 0))],
        out_shape=[jax.ShapeDtypeStruct((N_NODES, 128), jnp.int32),
                   jax.ShapeDtypeStruct((N_NODES, C_OUT), jnp.float32),
                   jax.ShapeDtypeStruct((N_EDGES // 8, 128), jnp.float32),
                   jax.ShapeDtypeStruct((N_NODES, C_OUT), jnp.float32)],
    )(node_feats, W2L, W2H, b2r, ea_wide, W1blk, b1w, root_W,
      bias.reshape(1, C_OUT))

    h_flat = hw.reshape(N_EDGES * C_OUT)
    p0, p1 = _sc_edge_kernel(gbp, bvals, h_flat, src, dst)

    bc = 2000
    out = pl.pallas_call(
        _combine_body,
        grid=(N_NODES // bc,),
        in_specs=[pl.BlockSpec((bc, C_OUT), lambda i: (i, 0)),
                  pl.BlockSpec((bc, C_OUT), lambda i: (i, 0)),
                  pl.BlockSpec((bc, C_OUT), lambda i: (i, 0))],
        out_specs=pl.BlockSpec((bc, C_OUT), lambda i: (i, 0)),
        out_shape=jax.ShapeDtypeStruct((N_NODES, C_OUT), jnp.float32),
    )(root, p0, p1)

    return out


# speedup vs baseline: 1.0430x; 1.0070x over previous
import functools

import jax
import jax.numpy as jnp
import numpy as np
from jax import lax
from jax.experimental import pallas as pl
from jax.experimental.pallas import tpu as pltpu
from jax.experimental.pallas import tpu_sc as plsc

N_NODES = 10000
N_EDGES = 160000
C_IN = 128
C_OUT = 16
EDGE_DIM = 16

NC = 2
NS = 16
NW = NC * NS
EDGES_PER_WORKER = N_EDGES // NW
CHUNK = 125
CP = 128
NCHUNKS = EDGES_PER_WORKER // CHUNK
ROWS_PER_TILE = 640
N_PAD = NS * ROWS_PER_TILE

_HI_MASK = np.int32(-65536)
_LO_MASK = np.int32(0xFFFF)


def _round_bf16_bits(x):
    b = lax.bitcast_convert_type(x, jnp.int32)
    return b + np.int32(0x7FFF) + ((b >> 16) & np.int32(1))


def _pre_body(nf_ref, w2l_ref, w2h_ref, b2_ref, ea_ref, w1_ref, b1_ref,
              rw_ref, b_ref, gbp_ref, bv_ref, h_ref, root_ref):
    nf = nf_ref[...]
    lo = jnp.dot(nf, w2l_ref[...], preferred_element_type=jnp.float32)
    hi = jnp.dot(nf, w2h_ref[...], preferred_element_type=jnp.float32)
    rl = _round_bf16_bits(lo)
    rh = _round_bf16_bits(hi)
    gbp_ref[...] = (rh & _HI_MASK) | ((rl >> 16) & _LO_MASK)
    bv_ref[...] = jnp.dot(nf, b2_ref[...], preferred_element_type=jnp.float32)
    h_ref[...] = jnp.maximum(
        jnp.dot(ea_ref[...], w1_ref[...],
                preferred_element_type=jnp.float32) + b1_ref[...], 0.0)
    root_ref[...] = jnp.dot(nf, rw_ref[...],
                            preferred_element_type=jnp.float32) + b_ref[...]


def _combine_body(root_ref, p0_ref, p1_ref, o_ref):
    o_ref[...] = root_ref[...] + p0_ref[...] + p1_ref[...]


_mesh = plsc.VectorSubcoreMesh(core_axis_name="c", subcore_axis_name="s")


@functools.partial(
    pl.kernel,
    mesh=_mesh,
    compiler_params=pltpu.CompilerParams(use_tc_tiling_on_sc=False,
                                         needs_layout_passes=False),
    out_type=(jax.ShapeDtypeStruct((N_PAD, C_OUT), jnp.float32),
              jax.ShapeDtypeStruct((N_PAD, C_OUT), jnp.float32)),
    scratch_types=[
        pltpu.VMEM((NCHUNKS, CP), jnp.int32),
        pltpu.VMEM((NCHUNKS, CP), jnp.int32),
        pltpu.VMEM((CP, 128), jnp.int32),
        pltpu.VMEM((CP, 128), jnp.int32),
        pltpu.VMEM((CP, C_OUT), jnp.float32),
        pltpu.VMEM((CP, C_OUT), jnp.float32),
        pltpu.VMEM((CHUNK * C_OUT,), jnp.float32),
        pltpu.VMEM((CHUNK * C_OUT,), jnp.float32),
        pltpu.VMEM((CP, C_OUT), jnp.float32),
        pltpu.VMEM((CP, C_OUT), jnp.float32),
        pltpu.VMEM((ROWS_PER_TILE, C_OUT), jnp.float32),
        pltpu.VMEM_SHARED((N_PAD, C_OUT), jnp.float32),
        pltpu.SemaphoreType.DMA,
        pltpu.SemaphoreType.DMA,
        pltpu.SemaphoreType.DMA,
        pltpu.SemaphoreType.DMA,
    ],
)
def _sc_edge_kernel(gbp_hbm, bv_hbm, h_hbm, src_hbm, dst_hbm,
                    out0_hbm, out1_hbm,
                    src_v, dst_v, rows0, rows1, bv0, bv1,
                    h0, h1, msg0, msg1, z_v, acc_sh,
                    semg0, semg1, sems0, sems1):
    c = lax.axis_index("c")
    s = lax.axis_index("s")
    wid = s * NC + c
    rows = (rows0, rows1)
    bv = (bv0, bv1)
    hb = (h0, h1)
    msg = (msg0, msg1)
    semg = (semg0, semg1)
    sems = (sems0, sems1)

    def _zero_row(i, carry):
        z_v[i, :] = jnp.zeros((C_OUT,), jnp.float32)
        return carry

    lax.fori_loop(0, ROWS_PER_TILE, _zero_row, 0)
    pltpu.sync_copy(z_v, acc_sh.at[pl.ds(s * ROWS_PER_TILE, ROWS_PER_TILE)])
    plsc.subcore_barrier()

    for b in (0, 1):
        for i in range(CHUNK, CP):
            msg[b][i, :] = jnp.zeros((C_OUT,), jnp.float32)

    pltpu.sync_copy(src_hbm.at[wid], src_v)
    pltpu.sync_copy(dst_hbm.at[wid], dst_v)

    hbase = wid * EDGES_PER_WORKER * C_OUT

    def _issue(j, b):
        idx = src_v.at[j]
        pltpu.async_copy(gbp_hbm.at[idx], rows[b], semg[b])
        pltpu.async_copy(bv_hbm.at[idx], bv[b], semg[b])
        pltpu.async_copy(h_hbm.at[pl.ds(hbase + j * (CHUNK * C_OUT),
                                        CHUNK * C_OUT)], hb[b], semg[b])

    def _drain(j, b):
        idx = src_v.at[j]
        pltpu.make_async_copy(gbp_hbm.at[idx], rows[b], semg[b]).wait()
        pltpu.make_async_copy(bv_hbm.at[idx], bv[b], semg[b]).wait()
        pltpu.make_async_copy(h_hbm.at[pl.ds(hbase + j * (CHUNK * C_OUT),
                                             CHUNK * C_OUT)],
                              hb[b], semg[b]).wait()

    def _process(t, j, b):
        _drain(j, b)

        @pl.when(t > 0)
        def _():
            pltpu.make_async_copy(msg[b], acc_sh.at[dst_v.at[j]],
                                  sems[b]).wait()

        def _edge(e, ecarry):
            hv = hb[b][pl.ds(e * C_OUT, C_OUT)]
            terms = [bv[b][e, :]]
            for q in range(8):
                v = rows[b][e, pl.ds(q * C_OUT, C_OUT)]
                lo = plsc.bitcast(v << 16, jnp.float32)
                hi = plsc.bitcast(v & _HI_MASK, jnp.float32)
                terms.append(hv[2 * q] * lo)
                terms.append(hv[2 * q + 1] * hi)
            while len(terms) > 1:
                nxt = [terms[i] + terms[i + 1]
                       for i in range(0, len(terms) - 1, 2)]
                if len(terms) % 2:
                    nxt.append(terms[-1])
                terms = nxt
            msg[b][e, :] = terms[0]
            return ecarry

        lax.fori_loop(0, CHUNK, _edge, 0, unroll=5)
        pltpu.async_copy(msg[b], acc_sh.at[dst_v.at[j]], sems[b], add=True)

    _issue(0, 0)

    def _super(t, carry):
        _issue(2 * t + 1, 1)
        _process(t, 2 * t, 0)

        @pl.when(t < NCHUNKS // 2 - 1)
        def _():
            _issue(2 * t + 2, 0)

        _process(t, 2 * t + 1, 1)
        return carry

    lax.fori_loop(0, NCHUNKS // 2, _super, 0)

    pltpu.make_async_copy(msg[0], acc_sh.at[dst_v.at[0]], sems[0]).wait()
    pltpu.make_async_copy(msg[1], acc_sh.at[dst_v.at[0]], sems[1]).wait()
    plsc.subcore_barrier()

    row0 = s * ROWS_PER_TILE

    @pl.when(c == 0)
    def _():
        pltpu.sync_copy(acc_sh.at[pl.ds(row0, ROWS_PER_TILE)],
                        out0_hbm.at[pl.ds(row0, ROWS_PER_TILE)])

    @pl.when(c == 1)
    def _():
        pltpu.sync_copy(acc_sh.at[pl.ds(row0, ROWS_PER_TILE)],
                        out1_hbm.at[pl.ds(row0, ROWS_PER_TILE)])


def kernel(node_feats, edge_attr, W1, b1, W2, b2, root_W, bias, edge_index):
    w2t = W2.reshape(EDGE_DIM, C_IN, C_OUT).transpose(1, 0, 2)
    W2L = w2t[:, 0::2, :].reshape(C_IN, 128)
    W2H = w2t[:, 1::2, :].reshape(C_IN, 128)
    b2r = b2.reshape(C_IN, C_OUT)

    src = jnp.pad(edge_index[0].reshape(NW, NCHUNKS, CHUNK),
                  ((0, 0), (0, 0), (0, CP - CHUNK)))
    dst = jnp.pad(edge_index[1].reshape(NW, NCHUNKS, CHUNK),
                  ((0, 0), (0, 0), (0, CP - CHUNK)))

    ea_wide = edge_attr.reshape(N_EDGES // 8, 8 * EDGE_DIM)
    W1blk = jnp.kron(jnp.eye(8, dtype=W1.dtype), W1)
    b1w = jnp.tile(b1, 8).reshape(1, 8 * C_OUT)

    ng = 5
    bn = N_NODES // ng
    bew = N_EDGES // 8 // ng
    gbp, bvals, hw, root = pl.pallas_call(
        _pre_body,
        grid=(ng,),
        in_specs=[pl.BlockSpec((bn, C_IN), lambda i: (i, 0)),
                  pl.BlockSpec((C_IN, 128), lambda i: (0, 0)),
                  pl.BlockSpec((C_IN, 128), lambda i: (0, 0)),
                  pl.BlockSpec((C_IN, C_OUT), lambda i: (0, 0)),
                  pl.BlockSpec((bew, 128), lambda i: (i, 0)),
                  pl.BlockSpec((128, 128), lambda i: (0, 0)),
                  pl.BlockSpec((1, 128), lambda i: (0, 0)),
                  pl.BlockSpec((C_IN, C_OUT), lambda i: (0, 0)),
                  pl.BlockSpec((1, C_OUT), lambda i: (0, 0))],
        out_specs=[pl.BlockSpec((bn, 128), lambda i: (i, 0)),
                   pl.BlockSpec((bn, C_OUT), lambda i: (i, 0)),
                   pl.BlockSpec((bew, 128), lambda i: (i, 0)),
                   pl.BlockSpec((bn, C_OUT), lambda i: (i, 0))],
        out_shape=[jax.ShapeDtypeStruct((N_NODES, 128), jnp.int32),
                   jax.ShapeDtypeStruct((N_NODES, C_OUT), jnp.float32),
                   jax.ShapeDtypeStruct((N_EDGES // 8, 128), jnp.float32),
                   jax.ShapeDtypeStruct((N_NODES, C_OUT), jnp.float32)],
    )(node_feats, W2L, W2H, b2r, ea_wide, W1blk, b1w, root_W,
      bias.reshape(1, C_OUT))

    h_flat = hw.reshape(N_EDGES * C_OUT)
    p0, p1 = _sc_edge_kernel(gbp, bvals, h_flat, src, dst)

    bc = 2000
    out = pl.pallas_call(
        _combine_body,
        grid=(N_NODES // bc,),
        in_specs=[pl.BlockSpec((bc, C_OUT), lambda i: (i, 0)),
                  pl.BlockSpec((bc, C_OUT), lambda i: (i, 0)),
                  pl.BlockSpec((bc, C_OUT), lambda i: (i, 0))],
        out_specs=pl.BlockSpec((bc, C_OUT), lambda i: (i, 0)),
        out_shape=jax.ShapeDtypeStruct((N_NODES, C_OUT), jnp.float32),
    )(root, p0, p1)

    return out


# speedup vs baseline: 1.2836x; 1.2307x over previous
import functools

import jax
import jax.numpy as jnp
from jax import lax
from jax.experimental import pallas as pl
from jax.experimental.pallas import tpu as pltpu
from jax.experimental.pallas import tpu_sc as plsc

N_NODES = 10000
N_EDGES = 160000
C_IN = 128
C_OUT = 16
EDGE_DIM = 16

NC = 2
NS = 16
NW = NC * NS
EDGES_PER_WORKER = N_EDGES // NW
CHUNK = 125
NCHUNKS = EDGES_PER_WORKER // CHUNK
ROWS_PER_TILE = 640
N_PAD = NS * ROWS_PER_TILE


def _pre_body(nf_ref, w2a_ref, w2b_ref, b2_ref, ea_ref, w1_ref, b1_ref,
              rw_ref, b_ref, gba_ref, gbb_ref, bv_ref, h_ref, root_ref):
    nf = nf_ref[...]
    gba_ref[...] = jnp.dot(nf, w2a_ref[...], preferred_element_type=jnp.float32)
    gbb_ref[...] = jnp.dot(nf, w2b_ref[...], preferred_element_type=jnp.float32)
    bv_ref[...] = jnp.dot(nf, b2_ref[...], preferred_element_type=jnp.float32)
    h_ref[...] = jnp.maximum(
        jnp.dot(ea_ref[...], w1_ref[...],
                preferred_element_type=jnp.float32) + b1_ref[...], 0.0)
    root_ref[...] = jnp.dot(nf, rw_ref[...],
                            preferred_element_type=jnp.float32) + b_ref[...]


def _combine_body(root_ref, p0_ref, p1_ref, o_ref):
    o_ref[...] = root_ref[...] + p0_ref[...] + p1_ref[...]


_mesh = plsc.VectorSubcoreMesh(core_axis_name="c", subcore_axis_name="s")


@functools.partial(
    pl.kernel,
    mesh=_mesh,
    compiler_params=pltpu.CompilerParams(use_tc_tiling_on_sc=False),
    out_type=(jax.ShapeDtypeStruct((N_PAD, C_OUT), jnp.float32),
              jax.ShapeDtypeStruct((N_PAD, C_OUT), jnp.float32)),
    scratch_types=[
        pltpu.VMEM((NCHUNKS, CHUNK), jnp.int32),
        pltpu.VMEM((NCHUNKS, CHUNK), jnp.int32),
        pltpu.VMEM((CHUNK, 128), jnp.float32),
        pltpu.VMEM((CHUNK, 128), jnp.float32),
        pltpu.VMEM((CHUNK, 128), jnp.float32),
        pltpu.VMEM((CHUNK, 128), jnp.float32),
        pltpu.VMEM((CHUNK, C_OUT), jnp.float32),
        pltpu.VMEM((CHUNK, C_OUT), jnp.float32),
        pltpu.VMEM((CHUNK * C_OUT,), jnp.float32),
        pltpu.VMEM((CHUNK * C_OUT,), jnp.float32),
        pltpu.VMEM((CHUNK, C_OUT), jnp.float32),
        pltpu.VMEM((CHUNK, C_OUT), jnp.float32),
        pltpu.VMEM((ROWS_PER_TILE, C_OUT), jnp.float32),
        pltpu.VMEM_SHARED((N_PAD, C_OUT), jnp.float32),
        pltpu.SemaphoreType.DMA,
        pltpu.SemaphoreType.DMA,
        pltpu.SemaphoreType.DMA,
        pltpu.SemaphoreType.DMA,
    ],
)
def _sc_edge_kernel(gba_hbm, gbb_hbm, bv_hbm, h_hbm, src_hbm, dst_hbm,
                    out0_hbm, out1_hbm,
                    src_v, dst_v, rowsa0, rowsa1, rowsb0, rowsb1, bv0, bv1,
                    h0, h1, msg0, msg1, z_v, acc_sh,
                    semg0, semg1, sems0, sems1):
    c = lax.axis_index("c")
    s = lax.axis_index("s")
    wid = s * NC + c
    rowsa = (rowsa0, rowsa1)
    rowsb = (rowsb0, rowsb1)
    bv = (bv0, bv1)
    hb = (h0, h1)
    msg = (msg0, msg1)
    semg = (semg0, semg1)
    sems = (sems0, sems1)

    def _zero_row(i, carry):
        z_v[i, :] = jnp.zeros((C_OUT,), jnp.float32)
        return carry

    lax.fori_loop(0, ROWS_PER_TILE, _zero_row, 0)
    pltpu.sync_copy(z_v, acc_sh.at[pl.ds(s * ROWS_PER_TILE, ROWS_PER_TILE)])
    plsc.subcore_barrier()

    pltpu.sync_copy(src_hbm.at[wid], src_v)
    pltpu.sync_copy(dst_hbm.at[wid], dst_v)

    hbase = wid * EDGES_PER_WORKER * C_OUT

    def _issue(j, b):
        idx = src_v.at[j]
        pltpu.async_copy(gba_hbm.at[idx], rowsa[b], semg[b])
        pltpu.async_copy(gbb_hbm.at[idx], rowsb[b], semg[b])
        pltpu.async_copy(bv_hbm.at[idx], bv[b], semg[b])
        pltpu.async_copy(h_hbm.at[pl.ds(hbase + j * (CHUNK * C_OUT),
                                        CHUNK * C_OUT)], hb[b], semg[b])

    def _drain(j, b):
        idx = src_v.at[j]
        pltpu.make_async_copy(gba_hbm.at[idx], rowsa[b], semg[b]).wait()
        pltpu.make_async_copy(gbb_hbm.at[idx], rowsb[b], semg[b]).wait()
        pltpu.make_async_copy(bv_hbm.at[idx], bv[b], semg[b]).wait()
        pltpu.make_async_copy(h_hbm.at[pl.ds(hbase + j * (CHUNK * C_OUT),
                                             CHUNK * C_OUT)],
                              hb[b], semg[b]).wait()

    def _process(t, j, b):
        _drain(j, b)

        @pl.when(t > 0)
        def _():
            pltpu.make_async_copy(msg[b], acc_sh.at[dst_v.at[j]],
                                  sems[b]).wait()

        def _edge(e, ecarry):
            hv = hb[b][pl.ds(e * C_OUT, C_OUT)]
            acc = bv[b][e, :]
            for k in range(8):
                acc = acc + hv[k] * rowsa[b][e, pl.ds(k * C_OUT, C_OUT)]
            for k in range(8):
                acc = acc + hv[k + 8] * rowsb[b][e, pl.ds(k * C_OUT, C_OUT)]
            msg[b][e, :] = acc
            return ecarry

        lax.fori_loop(0, CHUNK, _edge, 0, unroll=5)
        pltpu.async_copy(msg[b], acc_sh.at[dst_v.at[j]], sems[b], add=True)

    _issue(0, 0)

    def _super(t, carry):
        _issue(2 * t + 1, 1)
        _process(t, 2 * t, 0)

        @pl.when(t < NCHUNKS // 2 - 1)
        def _():
            _issue(2 * t + 2, 0)

        _process(t, 2 * t + 1, 1)
        return carry

    lax.fori_loop(0, NCHUNKS // 2, _super, 0)

    pltpu.make_async_copy(msg[0], acc_sh.at[dst_v.at[0]], sems[0]).wait()
    pltpu.make_async_copy(msg[1], acc_sh.at[dst_v.at[0]], sems[1]).wait()
    plsc.subcore_barrier()

    row0 = s * ROWS_PER_TILE

    @pl.when(c == 0)
    def _():
        pltpu.sync_copy(acc_sh.at[pl.ds(row0, ROWS_PER_TILE)],
                        out0_hbm.at[pl.ds(row0, ROWS_PER_TILE)])

    @pl.when(c == 1)
    def _():
        pltpu.sync_copy(acc_sh.at[pl.ds(row0, ROWS_PER_TILE)],
                        out1_hbm.at[pl.ds(row0, ROWS_PER_TILE)])


def kernel(node_feats, edge_attr, W1, b1, W2, b2, root_W, bias, edge_index):
    w2t = W2.reshape(EDGE_DIM, C_IN, C_OUT).transpose(1, 0, 2)
    W2a = w2t[:, :8, :].reshape(C_IN, 128)
    W2b = w2t[:, 8:, :].reshape(C_IN, 128)
    b2r = b2.reshape(C_IN, C_OUT)

    src = edge_index[0].reshape(NW, NCHUNKS, CHUNK)
    dst = edge_index[1].reshape(NW, NCHUNKS, CHUNK)

    ea_wide = edge_attr.reshape(N_EDGES // 8, 8 * EDGE_DIM)
    W1blk = jnp.kron(jnp.eye(8, dtype=W1.dtype), W1)
    b1w = jnp.tile(b1, 8).reshape(1, 8 * C_OUT)

    ng = 10
    bn = N_NODES // ng
    be = N_EDGES // ng
    gba, gbb, bvals, hw, root = pl.pallas_call(
        _pre_body,
        grid=(ng,),
        in_specs=[pl.BlockSpec((bn, C_IN), lambda i: (i, 0)),
                  pl.BlockSpec((C_IN, 128), lambda i: (0, 0)),
                  pl.BlockSpec((C_IN, 128), lambda i: (0, 0)),
                  pl.BlockSpec((C_IN, C_OUT), lambda i: (0, 0)),
                  pl.BlockSpec((be // 8, 128), lambda i: (i, 0)),
                  pl.BlockSpec((128, 128), lambda i: (0, 0)),
                  pl.BlockSpec((1, 128), lambda i: (0, 0)),
                  pl.BlockSpec((C_IN, C_OUT), lambda i: (0, 0)),
                  pl.BlockSpec((1, C_OUT), lambda i: (0, 0))],
        out_specs=[pl.BlockSpec((bn, 128), lambda i: (i, 0)),
                   pl.BlockSpec((bn, 128), lambda i: (i, 0)),
                   pl.BlockSpec((bn, C_OUT), lambda i: (i, 0)),
                   pl.BlockSpec((be // 8, 128), lambda i: (i, 0)),
                   pl.BlockSpec((bn, C_OUT), lambda i: (i, 0))],
        out_shape=[jax.ShapeDtypeStruct((N_NODES, 128), jnp.float32),
                   jax.ShapeDtypeStruct((N_NODES, 128), jnp.float32),
                   jax.ShapeDtypeStruct((N_NODES, C_OUT), jnp.float32),
                   jax.ShapeDtypeStruct((N_EDGES // 8, 128), jnp.float32),
                   jax.ShapeDtypeStruct((N_NODES, C_OUT), jnp.float32)],
    )(node_feats, W2a, W2b, b2r, ea_wide, W1blk, b1w, root_W,
      bias.reshape(1, C_OUT))

    h_flat = hw.reshape(N_EDGES * C_OUT)
    p0, p1 = _sc_edge_kernel(gba, gbb, bvals, h_flat, src, dst)
    p0 = p0[:N_NODES]
    p1 = p1[:N_NODES]

    bc = 2000
    out = pl.pallas_call(
        _combine_body,
        grid=(N_NODES // bc,),
        in_specs=[pl.BlockSpec((bc, C_OUT), lambda i: (i, 0)),
                  pl.BlockSpec((bc, C_OUT), lambda i: (i, 0)),
                  pl.BlockSpec((bc, C_OUT), lambda i: (i, 0))],
        out_specs=pl.BlockSpec((bc, C_OUT), lambda i: (i, 0)),
        out_shape=jax.ShapeDtypeStruct((N_NODES, C_OUT), jnp.float32),
    )(root, p0, p1)

    return out


# speedup vs baseline: 1.4073x; 1.0963x over previous
import functools

import jax
import jax.numpy as jnp
from jax import lax
from jax.experimental import pallas as pl
from jax.experimental.pallas import tpu as pltpu
from jax.experimental.pallas import tpu_sc as plsc

N_NODES = 10000
N_EDGES = 160000
C_IN = 128
C_OUT = 16
EDGE_DIM = 16

NC = 2
NS = 16
NW = NC * NS
EDGES_PER_WORKER = N_EDGES // NW
CHUNK = 125
NCHUNKS = EDGES_PER_WORKER // CHUNK
ROWS_PER_TILE = 640
N_PAD = NS * ROWS_PER_TILE


def _pre_body(nf_ref, w2a_ref, w2b_ref, b2_ref, ea_ref, w1_ref, b1_ref,
              rw_ref, b_ref, gba_ref, gbb_ref, bv_ref, h_ref, root_ref):
    nf = nf_ref[...]
    gba_ref[...] = jnp.dot(nf, w2a_ref[...], preferred_element_type=jnp.float32)
    gbb_ref[...] = jnp.dot(nf, w2b_ref[...], preferred_element_type=jnp.float32)
    bv_ref[...] = jnp.dot(nf, b2_ref[...], preferred_element_type=jnp.float32)
    h_ref[...] = jnp.maximum(
        jnp.dot(ea_ref[...], w1_ref[...],
                preferred_element_type=jnp.float32) + b1_ref[...], 0.0)
    root_ref[...] = jnp.dot(nf, rw_ref[...],
                            preferred_element_type=jnp.float32) + b_ref[...]


def _combine_body(root_ref, p0_ref, p1_ref, o_ref):
    o_ref[...] = root_ref[...] + p0_ref[...] + p1_ref[...]


_mesh = plsc.VectorSubcoreMesh(core_axis_name="c", subcore_axis_name="s")


@functools.partial(
    pl.kernel,
    mesh=_mesh,
    compiler_params=pltpu.CompilerParams(use_tc_tiling_on_sc=False),
    out_type=(jax.ShapeDtypeStruct((N_PAD, C_OUT), jnp.float32),
              jax.ShapeDtypeStruct((N_PAD, C_OUT), jnp.float32)),
    scratch_types=[
        pltpu.VMEM((NCHUNKS, CHUNK), jnp.int32),
        pltpu.VMEM((NCHUNKS, CHUNK), jnp.int32),
        pltpu.VMEM((CHUNK, 128), jnp.float32),
        pltpu.VMEM((CHUNK, 128), jnp.float32),
        pltpu.VMEM((CHUNK, 128), jnp.float32),
        pltpu.VMEM((CHUNK, 128), jnp.float32),
        pltpu.VMEM((CHUNK, C_OUT), jnp.float32),
        pltpu.VMEM((CHUNK, C_OUT), jnp.float32),
        pltpu.VMEM((CHUNK * C_OUT,), jnp.float32),
        pltpu.VMEM((CHUNK * C_OUT,), jnp.float32),
        pltpu.VMEM((CHUNK, C_OUT), jnp.float32),
        pltpu.VMEM((CHUNK, C_OUT), jnp.float32),
        pltpu.VMEM((ROWS_PER_TILE, C_OUT), jnp.float32),
        pltpu.VMEM_SHARED((N_PAD, C_OUT), jnp.float32),
        pltpu.SemaphoreType.DMA,
        pltpu.SemaphoreType.DMA,
        pltpu.SemaphoreType.DMA,
        pltpu.SemaphoreType.DMA,
    ],
)
def _sc_edge_kernel(gba_hbm, gbb_hbm, bv_hbm, h_hbm, src_hbm, dst_hbm,
                    out0_hbm, out1_hbm,
                    src_v, dst_v, rowsa0, rowsa1, rowsb0, rowsb1, bv0, bv1,
                    h0, h1, msg0, msg1, z_v, acc_sh,
                    semg0, semg1, sems0, sems1):
    c = lax.axis_index("c")
    s = lax.axis_index("s")
    wid = s * NC + c
    rowsa = (rowsa0, rowsa1)
    rowsb = (rowsb0, rowsb1)
    bv = (bv0, bv1)
    hb = (h0, h1)
    msg = (msg0, msg1)
    semg = (semg0, semg1)
    sems = (sems0, sems1)

    def _zero_row(i, carry):
        z_v[i, :] = jnp.zeros((C_OUT,), jnp.float32)
        return carry

    lax.fori_loop(0, ROWS_PER_TILE, _zero_row, 0)
    pltpu.sync_copy(z_v, acc_sh.at[pl.ds(s * ROWS_PER_TILE, ROWS_PER_TILE)])
    plsc.subcore_barrier()

    pltpu.sync_copy(src_hbm.at[wid], src_v)
    pltpu.sync_copy(dst_hbm.at[wid], dst_v)

    hbase = wid * EDGES_PER_WORKER * C_OUT

    def _issue(j, b):
        idx = src_v.at[j]
        pltpu.async_copy(gba_hbm.at[idx], rowsa[b], semg[b])
        pltpu.async_copy(gbb_hbm.at[idx], rowsb[b], semg[b])
        pltpu.async_copy(bv_hbm.at[idx], bv[b], semg[b])
        pltpu.async_copy(h_hbm.at[pl.ds(hbase + j * (CHUNK * C_OUT),
                                        CHUNK * C_OUT)], hb[b], semg[b])

    def _drain(j, b):
        idx = src_v.at[j]
        pltpu.make_async_copy(gba_hbm.at[idx], rowsa[b], semg[b]).wait()
        pltpu.make_async_copy(gbb_hbm.at[idx], rowsb[b], semg[b]).wait()
        pltpu.make_async_copy(bv_hbm.at[idx], bv[b], semg[b]).wait()
        pltpu.make_async_copy(h_hbm.at[pl.ds(hbase + j * (CHUNK * C_OUT),
                                             CHUNK * C_OUT)],
                              hb[b], semg[b]).wait()

    def _process(t, j, b):
        _drain(j, b)

        @pl.when(t > 0)
        def _():
            pltpu.make_async_copy(msg[b], acc_sh.at[dst_v.at[j]],
                                  sems[b]).wait()

        def _edge(e, ecarry):
            hv = hb[b][pl.ds(e * C_OUT, C_OUT)]
            terms = [bv[b][e, :]]
            for k in range(8):
                terms.append(hv[k] * rowsa[b][e, pl.ds(k * C_OUT, C_OUT)])
            for k in range(8):
                terms.append(
                    hv[k + 8] * rowsb[b][e, pl.ds(k * C_OUT, C_OUT)])
            while len(terms) > 1:
                nxt = [terms[i] + terms[i + 1]
                       for i in range(0, len(terms) - 1, 2)]
                if len(terms) % 2:
                    nxt.append(terms[-1])
                terms = nxt
            msg[b][e, :] = terms[0]
            return ecarry

        lax.fori_loop(0, CHUNK, _edge, 0, unroll=5)
        pltpu.async_copy(msg[b], acc_sh.at[dst_v.at[j]], sems[b], add=True)

    _issue(0, 0)

    def _super(t, carry):
        _issue(2 * t + 1, 1)
        _process(t, 2 * t, 0)

        @pl.when(t < NCHUNKS // 2 - 1)
        def _():
            _issue(2 * t + 2, 0)

        _process(t, 2 * t + 1, 1)
        return carry

    lax.fori_loop(0, NCHUNKS // 2, _super, 0)

    pltpu.make_async_copy(msg[0], acc_sh.at[dst_v.at[0]], sems[0]).wait()
    pltpu.make_async_copy(msg[1], acc_sh.at[dst_v.at[0]], sems[1]).wait()
    plsc.subcore_barrier()

    row0 = s * ROWS_PER_TILE

    @pl.when(c == 0)
    def _():
        pltpu.sync_copy(acc_sh.at[pl.ds(row0, ROWS_PER_TILE)],
                        out0_hbm.at[pl.ds(row0, ROWS_PER_TILE)])

    @pl.when(c == 1)
    def _():
        pltpu.sync_copy(acc_sh.at[pl.ds(row0, ROWS_PER_TILE)],
                        out1_hbm.at[pl.ds(row0, ROWS_PER_TILE)])


def kernel(node_feats, edge_attr, W1, b1, W2, b2, root_W, bias, edge_index):
    w2t = W2.reshape(EDGE_DIM, C_IN, C_OUT).transpose(1, 0, 2)
    W2a = w2t[:, :8, :].reshape(C_IN, 128)
    W2b = w2t[:, 8:, :].reshape(C_IN, 128)
    b2r = b2.reshape(C_IN, C_OUT)

    src = edge_index[0].reshape(NW, NCHUNKS, CHUNK)
    dst = edge_index[1].reshape(NW, NCHUNKS, CHUNK)

    ea_wide = edge_attr.reshape(N_EDGES // 8, 8 * EDGE_DIM)
    W1blk = jnp.kron(jnp.eye(8, dtype=W1.dtype), W1)
    b1w = jnp.tile(b1, 8).reshape(1, 8 * C_OUT)

    ng = 10
    bn = N_NODES // ng
    be = N_EDGES // ng
    gba, gbb, bvals, hw, root = pl.pallas_call(
        _pre_body,
        grid=(ng,),
        in_specs=[pl.BlockSpec((bn, C_IN), lambda i: (i, 0)),
                  pl.BlockSpec((C_IN, 128), lambda i: (0, 0)),
                  pl.BlockSpec((C_IN, 128), lambda i: (0, 0)),
                  pl.BlockSpec((C_IN, C_OUT), lambda i: (0, 0)),
                  pl.BlockSpec((be // 8, 128), lambda i: (i, 0)),
                  pl.BlockSpec((128, 128), lambda i: (0, 0)),
                  pl.BlockSpec((1, 128), lambda i: (0, 0)),
                  pl.BlockSpec((C_IN, C_OUT), lambda i: (0, 0)),
                  pl.BlockSpec((1, C_OUT), lambda i: (0, 0))],
        out_specs=[pl.BlockSpec((bn, 128), lambda i: (i, 0)),
                   pl.BlockSpec((bn, 128), lambda i: (i, 0)),
                   pl.BlockSpec((bn, C_OUT), lambda i: (i, 0)),
                   pl.BlockSpec((be // 8, 128), lambda i: (i, 0)),
                   pl.BlockSpec((bn, C_OUT), lambda i: (i, 0))],
        out_shape=[jax.ShapeDtypeStruct((N_NODES, 128), jnp.float32),
                   jax.ShapeDtypeStruct((N_NODES, 128), jnp.float32),
                   jax.ShapeDtypeStruct((N_NODES, C_OUT), jnp.float32),
                   jax.ShapeDtypeStruct((N_EDGES // 8, 128), jnp.float32),
                   jax.ShapeDtypeStruct((N_NODES, C_OUT), jnp.float32)],
    )(node_feats, W2a, W2b, b2r, ea_wide, W1blk, b1w, root_W,
      bias.reshape(1, C_OUT))

    h_flat = hw.reshape(N_EDGES * C_OUT)
    p0, p1 = _sc_edge_kernel(gba, gbb, bvals, h_flat, src, dst)
    p0 = p0[:N_NODES]
    p1 = p1[:N_NODES]

    bc = 2000
    out = pl.pallas_call(
        _combine_body,
        grid=(N_NODES // bc,),
        in_specs=[pl.BlockSpec((bc, C_OUT), lambda i: (i, 0)),
                  pl.BlockSpec((bc, C_OUT), lambda i: (i, 0)),
                  pl.BlockSpec((bc, C_OUT), lambda i: (i, 0))],
        out_specs=pl.BlockSpec((bc, C_OUT), lambda i: (i, 0)),
        out_shape=jax.ShapeDtypeStruct((N_NODES, C_OUT), jnp.float32),
    )(root, p0, p1)

    return out


# speedup vs baseline: 1.4249x; 1.0126x over previous
import functools

import jax
import jax.numpy as jnp
from jax import lax
from jax.experimental import pallas as pl
from jax.experimental.pallas import tpu as pltpu
from jax.experimental.pallas import tpu_sc as plsc

N_NODES = 10000
N_EDGES = 160000
C_IN = 128
C_OUT = 16
EDGE_DIM = 16

NC = 2
NS = 16
NW = NC * NS
EDGES_PER_WORKER = N_EDGES // NW
CHUNK = 125
NCHUNKS = EDGES_PER_WORKER // CHUNK
ROWS_PER_TILE = 640
N_PAD = NS * ROWS_PER_TILE


def _pre_body(nf_ref, w2a_ref, w2b_ref, b2_ref, ea_ref, w1_ref, b1_ref,
              rw_ref, b_ref, gba_ref, gbb_ref, bv_ref, h_ref, root_ref):
    nf = nf_ref[...]
    gba_ref[...] = jnp.dot(nf, w2a_ref[...], preferred_element_type=jnp.float32)
    gbb_ref[...] = jnp.dot(nf, w2b_ref[...], preferred_element_type=jnp.float32)
    bv_ref[...] = jnp.dot(nf, b2_ref[...], preferred_element_type=jnp.float32)
    h_ref[...] = jnp.maximum(
        jnp.dot(ea_ref[...], w1_ref[...],
                preferred_element_type=jnp.float32) + b1_ref[...], 0.0)
    root_ref[...] = jnp.dot(nf, rw_ref[...],
                            preferred_element_type=jnp.float32) + b_ref[...]


def _combine_body(root_ref, p0_ref, p1_ref, o_ref):
    o_ref[...] = root_ref[...] + p0_ref[...] + p1_ref[...]


_mesh = plsc.VectorSubcoreMesh(core_axis_name="c", subcore_axis_name="s")


@functools.partial(
    pl.kernel,
    mesh=_mesh,
    compiler_params=pltpu.CompilerParams(use_tc_tiling_on_sc=False),
    out_type=(jax.ShapeDtypeStruct((N_PAD, C_OUT), jnp.float32),
              jax.ShapeDtypeStruct((N_PAD, C_OUT), jnp.float32)),
    scratch_types=[
        pltpu.VMEM((NCHUNKS, CHUNK), jnp.int32),
        pltpu.VMEM((NCHUNKS, CHUNK), jnp.int32),
        pltpu.VMEM((CHUNK, 128), jnp.float32),
        pltpu.VMEM((CHUNK, 128), jnp.float32),
        pltpu.VMEM((CHUNK, 128), jnp.float32),
        pltpu.VMEM((CHUNK, 128), jnp.float32),
        pltpu.VMEM((CHUNK, C_OUT), jnp.float32),
        pltpu.VMEM((CHUNK, C_OUT), jnp.float32),
        pltpu.VMEM((CHUNK * C_OUT,), jnp.float32),
        pltpu.VMEM((CHUNK * C_OUT,), jnp.float32),
        pltpu.VMEM((CHUNK, C_OUT), jnp.float32),
        pltpu.VMEM((CHUNK, C_OUT), jnp.float32),
        pltpu.VMEM((ROWS_PER_TILE, C_OUT), jnp.float32),
        pltpu.VMEM_SHARED((N_PAD, C_OUT), jnp.float32),
        pltpu.SemaphoreType.DMA,
        pltpu.SemaphoreType.DMA,
        pltpu.SemaphoreType.DMA,
        pltpu.SemaphoreType.DMA,
    ],
)
def _sc_edge_kernel(gba_hbm, gbb_hbm, bv_hbm, h_hbm, src_hbm, dst_hbm,
                    out0_hbm, out1_hbm,
                    src_v, dst_v, rowsa0, rowsa1, rowsb0, rowsb1, bv0, bv1,
                    h0, h1, msg0, msg1, z_v, acc_sh,
                    semg0, semg1, sems0, sems1):
    c = lax.axis_index("c")
    s = lax.axis_index("s")
    wid = s * NC + c
    rowsa = (rowsa0, rowsa1)
    rowsb = (rowsb0, rowsb1)
    bv = (bv0, bv1)
    hb = (h0, h1)
    msg = (msg0, msg1)
    semg = (semg0, semg1)
    sems = (sems0, sems1)

    def _zero_row(i, carry):
        z_v[i, :] = jnp.zeros((C_OUT,), jnp.float32)
        return carry

    lax.fori_loop(0, ROWS_PER_TILE, _zero_row, 0)
    pltpu.sync_copy(z_v, acc_sh.at[pl.ds(s * ROWS_PER_TILE, ROWS_PER_TILE)])
    plsc.subcore_barrier()

    pltpu.sync_copy(src_hbm.at[wid], src_v)
    pltpu.sync_copy(dst_hbm.at[wid], dst_v)

    hbase = wid * EDGES_PER_WORKER * C_OUT

    def _issue(j, b):
        idx = src_v.at[j]
        pltpu.async_copy(gba_hbm.at[idx], rowsa[b], semg[b])
        pltpu.async_copy(gbb_hbm.at[idx], rowsb[b], semg[b])
        pltpu.async_copy(bv_hbm.at[idx], bv[b], semg[b])
        pltpu.async_copy(h_hbm.at[pl.ds(hbase + j * (CHUNK * C_OUT),
                                        CHUNK * C_OUT)], hb[b], semg[b])

    def _drain(j, b):
        idx = src_v.at[j]
        pltpu.make_async_copy(gba_hbm.at[idx], rowsa[b], semg[b]).wait()
        pltpu.make_async_copy(gbb_hbm.at[idx], rowsb[b], semg[b]).wait()
        pltpu.make_async_copy(bv_hbm.at[idx], bv[b], semg[b]).wait()
        pltpu.make_async_copy(h_hbm.at[pl.ds(hbase + j * (CHUNK * C_OUT),
                                             CHUNK * C_OUT)],
                              hb[b], semg[b]).wait()

    def _process(t, j, b):
        _drain(j, b)

        @pl.when(t > 0)
        def _():
            pltpu.make_async_copy(msg[b], acc_sh.at[dst_v.at[j]],
                                  sems[b]).wait()

        def _edge(e, ecarry):
            hv = hb[b][pl.ds(e * C_OUT, C_OUT)]
            terms = [bv[b][e, :]]
            for k in range(8):
                terms.append(hv[k] * rowsa[b][e, pl.ds(k * C_OUT, C_OUT)])
            for k in range(8):
                terms.append(
                    hv[k + 8] * rowsb[b][e, pl.ds(k * C_OUT, C_OUT)])
            while len(terms) > 1:
                nxt = [terms[i] + terms[i + 1]
                       for i in range(0, len(terms) - 1, 2)]
                if len(terms) % 2:
                    nxt.append(terms[-1])
                terms = nxt
            msg[b][e, :] = terms[0]
            return ecarry

        lax.fori_loop(0, CHUNK, _edge, 0, unroll=5)
        pltpu.async_copy(msg[b], acc_sh.at[dst_v.at[j]], sems[b], add=True)

    _issue(0, 0)

    def _super(t, carry):
        _issue(2 * t + 1, 1)
        _process(t, 2 * t, 0)

        @pl.when(t < NCHUNKS // 2 - 1)
        def _():
            _issue(2 * t + 2, 0)

        _process(t, 2 * t + 1, 1)
        return carry

    lax.fori_loop(0, NCHUNKS // 2, _super, 0)

    pltpu.make_async_copy(msg[0], acc_sh.at[dst_v.at[0]], sems[0]).wait()
    pltpu.make_async_copy(msg[1], acc_sh.at[dst_v.at[0]], sems[1]).wait()
    plsc.subcore_barrier()

    row0 = s * ROWS_PER_TILE

    @pl.when(c == 0)
    def _():
        pltpu.sync_copy(acc_sh.at[pl.ds(row0, ROWS_PER_TILE)],
                        out0_hbm.at[pl.ds(row0, ROWS_PER_TILE)])

    @pl.when(c == 1)
    def _():
        pltpu.sync_copy(acc_sh.at[pl.ds(row0, ROWS_PER_TILE)],
                        out1_hbm.at[pl.ds(row0, ROWS_PER_TILE)])


def kernel(node_feats, edge_attr, W1, b1, W2, b2, root_W, bias, edge_index):
    w2t = W2.reshape(EDGE_DIM, C_IN, C_OUT).transpose(1, 0, 2)
    W2a = w2t[:, :8, :].reshape(C_IN, 128)
    W2b = w2t[:, 8:, :].reshape(C_IN, 128)
    b2r = b2.reshape(C_IN, C_OUT)

    src = edge_index[0].reshape(NW, NCHUNKS, CHUNK)
    dst = edge_index[1].reshape(NW, NCHUNKS, CHUNK)

    ea_wide = edge_attr.reshape(N_EDGES // 8, 8 * EDGE_DIM)
    W1blk = jnp.kron(jnp.eye(8, dtype=W1.dtype), W1)
    b1w = jnp.tile(b1, 8).reshape(1, 8 * C_OUT)

    ng = 10
    bn = N_NODES // ng
    be = N_EDGES // ng
    gba, gbb, bvals, hw, root = pl.pallas_call(
        _pre_body,
        grid=(ng,),
        in_specs=[pl.BlockSpec((bn, C_IN), lambda i: (i, 0)),
                  pl.BlockSpec((C_IN, 128), lambda i: (0, 0)),
                  pl.BlockSpec((C_IN, 128), lambda i: (0, 0)),
                  pl.BlockSpec((C_IN, C_OUT), lambda i: (0, 0)),
                  pl.BlockSpec((be // 8, 128), lambda i: (i, 0)),
                  pl.BlockSpec((128, 128), lambda i: (0, 0)),
                  pl.BlockSpec((1, 128), lambda i: (0, 0)),
                  pl.BlockSpec((C_IN, C_OUT), lambda i: (0, 0)),
                  pl.BlockSpec((1, C_OUT), lambda i: (0, 0))],
        out_specs=[pl.BlockSpec((bn, 128), lambda i: (i, 0)),
                   pl.BlockSpec((bn, 128), lambda i: (i, 0)),
                   pl.BlockSpec((bn, C_OUT), lambda i: (i, 0)),
                   pl.BlockSpec((be // 8, 128), lambda i: (i, 0)),
                   pl.BlockSpec((bn, C_OUT), lambda i: (i, 0))],
        out_shape=[jax.ShapeDtypeStruct((N_NODES, 128), jnp.float32),
                   jax.ShapeDtypeStruct((N_NODES, 128), jnp.float32),
                   jax.ShapeDtypeStruct((N_NODES, C_OUT), jnp.float32),
                   jax.ShapeDtypeStruct((N_EDGES // 8, 128), jnp.float32),
                   jax.ShapeDtypeStruct((N_NODES, C_OUT), jnp.float32)],
    )(node_feats, W2a, W2b, b2r, ea_wide, W1blk, b1w, root_W,
      bias.reshape(1, C_OUT))

    h_flat = hw.reshape(N_EDGES * C_OUT)
    p0, p1 = _sc_edge_kernel(gba, gbb, bvals, h_flat, src, dst)

    bc = 2000
    out = pl.pallas_call(
        _combine_body,
        grid=(N_NODES // bc,),
        in_specs=[pl.BlockSpec((bc, C_OUT), lambda i: (i, 0)),
                  pl.BlockSpec((bc, C_OUT), lambda i: (i, 0)),
                  pl.BlockSpec((bc, C_OUT), lambda i: (i, 0))],
        out_specs=pl.BlockSpec((bc, C_OUT), lambda i: (i, 0)),
        out_shape=jax.ShapeDtypeStruct((N_NODES, C_OUT), jnp.float32),
    )(root, p0, p1)

    return out
